# baseline (speedup 1.0000x reference)
import functools

import jax
import jax.numpy as jnp
import numpy as np
from jax import lax
from jax.experimental import pallas as pl
from jax.experimental.pallas import tpu as pltpu
from jax.experimental.pallas import tpu_sc as plsc

N = 10000
E = 320000
D = 128
NC = 2
NS = 16
NW = NC * NS
EPW = E // NW
C = 128
NCHUNK = E // C
H = 64
JOBS = E // H
SLOTS = (JOBS + NW - 1) // NW
NPAD = 10240
PD = D // 2
INV_SQRT_D = float(1.0 / np.sqrt(D))

_PHI = (1.0 + np.sqrt(5.0)) / 2.0
_HARMONICS = np.array(
    [_PHI / 2, _PHI, _PHI * 1.5, _PHI * _PHI, _PHI * _PHI * 1.5, _PHI ** 3],
    dtype=np.float32,
)


def _qkv_body(f_ref, w_ref, b_ref, dist_ref, pw1_ref, pb1_ref, pw2_ref,
              pb2_ref, q_ref, k_ref, v_ref, eb_ref):
    qkv = (
        jnp.dot(f_ref[...], w_ref[...], preferred_element_type=jnp.float32)
        + b_ref[...]
    )
    q_ref[...] = qkv[:, :D]
    k_ref[...] = qkv[:, D:2 * D]
    v_ref[...] = qkv[:, 2 * D:]
    d = dist_ref[...]
    m = jnp.abs(d - _HARMONICS[0])
    for hk in _HARMONICS[1:]:
        m = jnp.minimum(m, jnp.abs(d - hk))
    w1 = pw1_ref[...]
    b1 = pb1_ref[...]
    w2 = pw2_ref[...]
    acc = jnp.zeros_like(d)
    for j in range(16):
        h = m * w1[0, j] + b1[0, j]
        h = h * jax.nn.sigmoid(h)
        acc = acc + h * w2[0, j]
    pw = jax.nn.sigmoid(acc + pb2_ref[0, 0])
    eb_ref[...] = jnp.exp(pw * 2.0)


def _qkv(features, Wcat, bcat, edge_dist, pw1, pb1, pw2, pb2):
    outs = pl.pallas_call(
        _qkv_body,
        out_shape=[jax.ShapeDtypeStruct((N, D), jnp.float32)] * 3
        + [jax.ShapeDtypeStruct((E // 512, 512), jnp.float32)],
    )(features, Wcat, bcat, edge_dist.reshape(E // 512, 512), pw1,
      pb1.reshape(1, 16), pw2.reshape(1, 16), pb2.reshape(1, 1))
    return outs


def _edge_sc(Q, K, V, src3, dst3, eb3):
    mesh = plsc.VectorSubcoreMesh(core_axis_name="c", subcore_axis_name="s")

    @functools.partial(
        pl.kernel,
        out_type=[
            jax.ShapeDtypeStruct((NC, N, D), jnp.float32),
            jax.ShapeDtypeStruct((NC, NPAD), jnp.float32),
        ],
        mesh=mesh,
        compiler_params=pltpu.CompilerParams(needs_layout_passes=False),
        scratch_types=[
            [pltpu.VMEM((1, H), jnp.int32) for _ in range(4)],
            [pltpu.VMEM((1, H), jnp.int32) for _ in range(4)],
            [pltpu.VMEM((1, H), jnp.float32) for _ in range(4)],
            [pltpu.VMEM((H, D), jnp.float32) for _ in range(2)],
            [pltpu.VMEM((H, D), jnp.float32) for _ in range(2)],
            pltpu.VMEM((H, D), jnp.float32),
            pltpu.VMEM((H * 16,), jnp.float32),
            [pltpu.VMEM((H,), jnp.float32) for _ in range(2)],
            [pltpu.VMEM((1, H), jnp.int32) for _ in range(2)],
            pltpu.VMEM((128,), jnp.float32),
            pltpu.VMEM_SHARED((N, D), jnp.float32),
            pltpu.VMEM_SHARED((NPAD,), jnp.float32),
            [pltpu.SemaphoreType.DMA for _ in range(4)],
            [pltpu.SemaphoreType.DMA for _ in range(2)],
            pltpu.SemaphoreType.DMA,
            pltpu.SemaphoreType.DMA,
            [pltpu.SemaphoreType.DMA for _ in range(2)],
        ],
    )
    def ek(q_hbm, k_hbm, v_hbm, src_hbm, dst_hbm, eb_hbm, agg_out, sums_out,
           srch, dsth, ebh, qrows, krows, vrows, attnpart, wbuf, dscat, zb1,
           agg_sh, sums_sh, idxsem, qksem, vsem, ssem, wsem):
        c = lax.axis_index("c")
        s = lax.axis_index("s")
        zero16 = jnp.zeros((16,), jnp.float32)
        wid = c * NS + s
        lanes = lax.iota(jnp.int32, 16)

        def zrow(i, carry):
            for j in range(8):
                vrows[i, pl.ds(j * 16, 16)] = zero16
            return carry
        lax.fori_loop(0, H, zrow, 0)
        for i in range(8):
            zb1[pl.ds(i * 16, 16)] = zero16

        @pl.when(s < 10)
        def _():
            for i in range(15):
                pltpu.sync_copy(vrows,
                                agg_sh.at[pl.ds(s * 1000 + i * 64, 64)])
            pltpu.sync_copy(vrows.at[pl.ds(0, 40)],
                            agg_sh.at[pl.ds(s * 1000 + 960, 40)])

        @pl.when(s < 8)
        def _():
            for i in range(10):
                pltpu.sync_copy(zb1, sums_sh.at[pl.ds(s * 1280 + i * 128, 128)])

        plsc.subcore_barrier()

        def jid_of(m):
            return m * NW + wid

        def issue_idx(m, i):
            @pl.when(jid_of(m) < JOBS)
            def _():
                j = jid_of(m)
                pltpu.async_copy(src_hbm.at[j], srch[i], idxsem[i])
                pltpu.async_copy(dst_hbm.at[j], dsth[i], idxsem[i])
                pltpu.async_copy(eb_hbm.at[j], ebh[i], idxsem[i])

        def issue_qk(m, i, p):
            @pl.when(jid_of(m) < JOBS)
            def _():
                j = jid_of(m)
                pltpu.make_async_copy(src_hbm.at[j], srch[i], idxsem[i]).wait()
                pltpu.make_async_copy(dst_hbm.at[j], dsth[i], idxsem[i]).wait()
                pltpu.make_async_copy(eb_hbm.at[j], ebh[i], idxsem[i]).wait()
                pltpu.async_copy(q_hbm.at[dsth[i].at[0]], qrows[p], qksem[p])
                pltpu.async_copy(k_hbm.at[srch[i].at[0]], krows[p], qksem[p])

        def issue_v(m, i):
            @pl.when(jid_of(m) < JOBS)
            def _():
                pltpu.async_copy(v_hbm.at[srch[i].at[0]], vrows, vsem)

        def compute(m, i, p):
            @pl.when(jid_of(m) < JOBS)
            def _():
                pltpu.make_async_copy(
                    q_hbm.at[dsth[i].at[0]], qrows[p], qksem[p]).wait()
                pltpu.make_async_copy(
                    k_hbm.at[srch[i].at[0]], krows[p], qksem[p]).wait()

                for k in range(H // 16):
                    dscat[p][0, pl.ds(k * 16, 16)] = (
                        dsth[i][0, pl.ds(k * 16, 16)])

                qr = qrows[p]
                kr = krows[p]

                @plsc.parallel_loop(0, H, unroll=4)
                def dot_body(e):
                    acc = qr[e, pl.ds(0, 16)] * kr[e, pl.ds(0, 16)]
                    for j in range(1, 8):
                        acc = acc + (qr[e, pl.ds(j * 16, 16)]
                                     * kr[e, pl.ds(j * 16, 16)])
                    attnpart[pl.ds(e * 16, 16)] = acc

                @plsc.parallel_loop(0, H // 16, unroll=2)
                def red_body(g):
                    gbase = g * 256
                    parts = [plsc.load_gather(attnpart, [gbase + lanes * 16 + j])
                             for j in range(16)]
                    while len(parts) > 1:
                        parts = [a + b for a, b in zip(parts[::2], parts[1::2])]
                    attn16 = parts[0] * INV_SQRT_D
                    w16 = jnp.exp(attn16) * ebh[i][0, pl.ds(g * 16, 16)]
                    wbuf[p][pl.ds(g * 16, 16)] = w16

                pltpu.make_async_copy(
                    v_hbm.at[srch[i].at[0]], vrows, vsem).wait()

                @plsc.parallel_loop(0, H, unroll=4)
                def scale_body(e):
                    wsp = plsc.load_gather(
                        wbuf[p], [jnp.zeros((16,), jnp.int32) + e])
                    for j in range(8):
                        vrows[e, pl.ds(j * 16, 16)] = (
                            vrows[e, pl.ds(j * 16, 16)] * wsp)

                cs = pltpu.async_copy(
                    vrows, agg_sh.at[dscat[p].at[0]], ssem, add=True)
                pltpu.async_copy(
                    wbuf[p], sums_sh.at[dscat[p].at[0]], wsem[p], add=True)
                cs.wait()

        def wait_sums(m, p):
            @pl.when((m >= 0) & (jid_of(m) < JOBS))
            def _():
                pltpu.make_async_copy(
                    wbuf[p], sums_sh.at[dscat[p].at[0]], wsem[p]).wait()

        for i in range(4):
            issue_idx(i, i)
        issue_qk(0, 0, 0)
        issue_v(0, 0)

        def outer(k2, carry):
            kb = k2 * 4
            for u in range(4):
                m = kb + u
                issue_qk(m + 1, (u + 1) % 4, (u + 1) % 2)
                compute(m, u, u % 2)
                wait_sums(m - 1, (u + 1) % 2)
                issue_v(m + 1, (u + 1) % 4)
                issue_idx(m + 4, u)
            return carry
        lax.fori_loop(0, (SLOTS + 3) // 4, outer, 0)

        plsc.subcore_barrier()

        @pl.when(s < 10)
        def _():
            for i in range(15):
                r0 = s * 1000 + i * 64
                pltpu.sync_copy(agg_sh.at[pl.ds(r0, 64)], vrows)
                pltpu.sync_copy(vrows, agg_out.at[c].at[pl.ds(r0, 64)])
            r0 = s * 1000 + 960
            pltpu.sync_copy(agg_sh.at[pl.ds(r0, 40)], vrows.at[pl.ds(0, 40)])
            pltpu.sync_copy(vrows.at[pl.ds(0, 40)],
                            agg_out.at[c].at[pl.ds(r0, 40)])

        @pl.when(s < 8)
        def _():
            for i in range(10):
                r0 = s * 1280 + i * 128
                pltpu.sync_copy(sums_sh.at[pl.ds(r0, 128)], zb1)
                pltpu.sync_copy(zb1, sums_out.at[c].at[pl.ds(r0, 128)])

    return ek(Q, K, V, src3, dst3, eb3)


def _final_body(agg_ref, sums_ref, f_ref, wo_ref, bo_ref, g_ref, b_ref,
                out_ref):
    sums = sums_ref[0] + sums_ref[1]
    inv = 1.0 / jnp.maximum(sums, 1e-8)
    out_nodes = (agg_ref[0] + agg_ref[1]) * inv[:, None]
    output = (
        jnp.dot(out_nodes, wo_ref[...], preferred_element_type=jnp.float32)
        + bo_ref[...]
    )
    res = f_ref[...] + output
    mean = jnp.mean(res, axis=-1, keepdims=True)
    var = jnp.mean((res - mean) ** 2, axis=-1, keepdims=True)
    normed = (res - mean) * lax.rsqrt(var + 1e-5)
    out_ref[...] = normed * g_ref[...] + b_ref[...]


def _final(agg, sums, features, Wo, bo, gamma, beta):
    return pl.pallas_call(
        _final_body,
        out_shape=jax.ShapeDtypeStruct((N, D), jnp.float32),
    )(agg, sums, features, Wo, bo, gamma, beta)


def kernel(features, edge_index, edge_dist, Wq, bq, Wk, bk, Wv, bv,
           pw1, pb1, pw2, pb2, Wo, bo, gamma, beta):
    src = edge_index[0]
    dst = edge_index[1]

    Wcat = jnp.concatenate([Wq, Wk, Wv], axis=1)
    bcat = jnp.concatenate([bq, bk, bv], axis=0)
    Q, K, V, eb2d = _qkv(features, Wcat, bcat, edge_dist, pw1, pb1, pw2, pb2)

    eb = eb2d.reshape(E)

    src3 = src.reshape(JOBS, 1, H)
    dst3 = dst.reshape(JOBS, 1, H)
    eb3 = eb.reshape(JOBS, 1, H)
    agg, sums = _edge_sc(Q, K, V, src3, dst3, eb3)
    sums = sums[:, :N]

    return _final(agg, sums, features, Wo, bo, gamma, beta)


# speedup vs baseline: 1.0100x; 1.0100x over previous
import functools

import jax
import jax.numpy as jnp
import numpy as np
from jax import lax
from jax.experimental import pallas as pl
from jax.experimental.pallas import tpu as pltpu
from jax.experimental.pallas import tpu_sc as plsc

N = 10000
E = 320000
D = 128
NC = 2
NS = 16
NW = NC * NS
EPW = E // NW
C = 128
NCHUNK = E // C
H = 64
JOBS = E // H
SLOTS = (JOBS + NW - 1) // NW
NPAD = 10240
PD = D // 2
INV_SQRT_D = float(1.0 / np.sqrt(D))

_PHI = (1.0 + np.sqrt(5.0)) / 2.0
_HARMONICS = np.array(
    [_PHI / 2, _PHI, _PHI * 1.5, _PHI * _PHI, _PHI * _PHI * 1.5, _PHI ** 3],
    dtype=np.float32,
)


def _qkv_body(f_ref, w_ref, b_ref, dist_ref, pw1_ref, pb1_ref, pw2_ref,
              pb2_ref, q_ref, k_ref, v_ref, eb_ref):
    qkv = (
        jnp.dot(f_ref[...], w_ref[...], preferred_element_type=jnp.float32)
        + b_ref[...]
    )
    q_ref[...] = qkv[:, :D]
    k_ref[...] = qkv[:, D:2 * D]
    v_ref[...] = qkv[:, 2 * D:]
    d = dist_ref[...]
    m = jnp.abs(d - _HARMONICS[0])
    for hk in _HARMONICS[1:]:
        m = jnp.minimum(m, jnp.abs(d - hk))
    w1 = pw1_ref[...]
    b1 = pb1_ref[...]
    w2 = pw2_ref[...]
    acc = jnp.zeros_like(d)
    for j in range(16):
        h = m * w1[0, j] + b1[0, j]
        h = h * jax.nn.sigmoid(h)
        acc = acc + h * w2[0, j]
    pw = jax.nn.sigmoid(acc + pb2_ref[0, 0])
    eb_ref[...] = jnp.exp(pw * 2.0)


def _qkv(features, Wcat, bcat, edge_dist, pw1, pb1, pw2, pb2):
    outs = pl.pallas_call(
        _qkv_body,
        out_shape=[jax.ShapeDtypeStruct((N, D), jnp.float32)] * 3
        + [jax.ShapeDtypeStruct((E // 512, 512), jnp.float32)],
    )(features, Wcat, bcat, edge_dist.reshape(E // 512, 512), pw1,
      pb1.reshape(1, 16), pw2.reshape(1, 16), pb2.reshape(1, 1))
    return outs


def _edge_sc(Q, K, V, src3, dst3, eb3):
    mesh = plsc.VectorSubcoreMesh(core_axis_name="c", subcore_axis_name="s")

    @functools.partial(
        pl.kernel,
        out_type=[
            jax.ShapeDtypeStruct((NC, N, D), jnp.float32),
            jax.ShapeDtypeStruct((NC, NPAD), jnp.float32),
        ],
        mesh=mesh,
        compiler_params=pltpu.CompilerParams(needs_layout_passes=False),
        scratch_types=[
            [pltpu.VMEM((1, H), jnp.int32) for _ in range(4)],
            [pltpu.VMEM((1, H), jnp.int32) for _ in range(4)],
            [pltpu.VMEM((1, H), jnp.float32) for _ in range(4)],
            [pltpu.VMEM((H, D), jnp.float32) for _ in range(2)],
            [pltpu.VMEM((H, D), jnp.float32) for _ in range(2)],
            pltpu.VMEM((H, D), jnp.float32),
            pltpu.VMEM((H * 16,), jnp.float32),
            pltpu.VMEM((H,), jnp.float32),
            pltpu.VMEM((128,), jnp.float32),
            pltpu.VMEM_SHARED((N, D), jnp.float32),
            pltpu.VMEM_SHARED((NPAD,), jnp.float32),
            [pltpu.SemaphoreType.DMA for _ in range(4)],
            [pltpu.SemaphoreType.DMA for _ in range(2)],
            pltpu.SemaphoreType.DMA,
            pltpu.SemaphoreType.DMA,
            pltpu.SemaphoreType.DMA,
        ],
    )
    def ek(q_hbm, k_hbm, v_hbm, src_hbm, dst_hbm, eb_hbm, agg_out, sums_out,
           srch, dsth, ebh, qrows, krows, vrows, attnpart, wbuf, zb1,
           agg_sh, sums_sh, idxsem, qksem, vsem, ssem, wsem):
        c = lax.axis_index("c")
        s = lax.axis_index("s")
        zero16 = jnp.zeros((16,), jnp.float32)
        wid = c * NS + s
        lanes = lax.iota(jnp.int32, 16)

        def zrow(i, carry):
            for j in range(8):
                vrows[i, pl.ds(j * 16, 16)] = zero16
            return carry
        lax.fori_loop(0, H, zrow, 0)
        for i in range(8):
            zb1[pl.ds(i * 16, 16)] = zero16

        @pl.when(s < 10)
        def _():
            for i in range(15):
                pltpu.sync_copy(vrows,
                                agg_sh.at[pl.ds(s * 1000 + i * 64, 64)])
            pltpu.sync_copy(vrows.at[pl.ds(0, 40)],
                            agg_sh.at[pl.ds(s * 1000 + 960, 40)])

        @pl.when(s < 8)
        def _():
            for i in range(10):
                pltpu.sync_copy(zb1, sums_sh.at[pl.ds(s * 1280 + i * 128, 128)])

        plsc.subcore_barrier()

        def jid_of(m):
            return m * NW + wid

        def issue_idx(m, i):
            @pl.when(jid_of(m) < JOBS)
            def _():
                j = jid_of(m)
                pltpu.async_copy(src_hbm.at[j], srch[i], idxsem[i])
                pltpu.async_copy(dst_hbm.at[j], dsth[i], idxsem[i])
                pltpu.async_copy(eb_hbm.at[j], ebh[i], idxsem[i])

        def issue_qk(m, i, p):
            @pl.when(jid_of(m) < JOBS)
            def _():
                j = jid_of(m)
                pltpu.make_async_copy(src_hbm.at[j], srch[i], idxsem[i]).wait()
                pltpu.make_async_copy(dst_hbm.at[j], dsth[i], idxsem[i]).wait()
                pltpu.make_async_copy(eb_hbm.at[j], ebh[i], idxsem[i]).wait()
                pltpu.async_copy(q_hbm.at[dsth[i].at[0]], qrows[p], qksem[p])
                pltpu.async_copy(k_hbm.at[srch[i].at[0]], krows[p], qksem[p])

        def issue_v(m, i):
            @pl.when(jid_of(m) < JOBS)
            def _():
                pltpu.async_copy(v_hbm.at[srch[i].at[0]], vrows, vsem)

        def compute(m, i, p):
            @pl.when(jid_of(m) < JOBS)
            def _():
                pltpu.make_async_copy(
                    q_hbm.at[dsth[i].at[0]], qrows[p], qksem[p]).wait()
                pltpu.make_async_copy(
                    k_hbm.at[srch[i].at[0]], krows[p], qksem[p]).wait()

                qr = qrows[p]
                kr = krows[p]

                @plsc.parallel_loop(0, H, unroll=4)
                def dot_body(e):
                    acc = qr[e, pl.ds(0, 16)] * kr[e, pl.ds(0, 16)]
                    for j in range(1, 8):
                        acc = acc + (qr[e, pl.ds(j * 16, 16)]
                                     * kr[e, pl.ds(j * 16, 16)])
                    attnpart[pl.ds(e * 16, 16)] = acc

                @plsc.parallel_loop(0, H // 16, unroll=2)
                def red_body(g):
                    gbase = g * 256
                    parts = [plsc.load_gather(attnpart, [gbase + lanes * 16 + j])
                             for j in range(16)]
                    while len(parts) > 1:
                        parts = [a + b for a, b in zip(parts[::2], parts[1::2])]
                    attn16 = parts[0] * INV_SQRT_D
                    w16 = jnp.exp(attn16) * ebh[i][0, pl.ds(g * 16, 16)]
                    wbuf[pl.ds(g * 16, 16)] = w16

                pltpu.make_async_copy(
                    v_hbm.at[srch[i].at[0]], vrows, vsem).wait()

                @plsc.parallel_loop(0, H, unroll=4)
                def scale_body(e):
                    wsp = plsc.load_gather(
                        wbuf, [jnp.zeros((16,), jnp.int32) + e])
                    for j in range(8):
                        vrows[e, pl.ds(j * 16, 16)] = (
                            vrows[e, pl.ds(j * 16, 16)] * wsp)

                cs = pltpu.async_copy(
                    vrows, agg_sh.at[dsth[i].at[0]], ssem, add=True)
                cw = pltpu.async_copy(
                    wbuf, sums_sh.at[dsth[i].at[0]], wsem, add=True)
                cs.wait()
                cw.wait()

        for i in range(4):
            issue_idx(i, i)
        issue_qk(0, 0, 0)
        issue_v(0, 0)

        def outer(k2, carry):
            kb = k2 * 4
            for u in range(4):
                m = kb + u
                issue_qk(m + 1, (u + 1) % 4, (u + 1) % 2)
                compute(m, u, u % 2)
                issue_v(m + 1, (u + 1) % 4)
                issue_idx(m + 4, u)
            return carry
        lax.fori_loop(0, (SLOTS + 3) // 4, outer, 0)

        plsc.subcore_barrier()

        @pl.when(s < 10)
        def _():
            for i in range(15):
                r0 = s * 1000 + i * 64
                pltpu.sync_copy(agg_sh.at[pl.ds(r0, 64)], vrows)
                pltpu.sync_copy(vrows, agg_out.at[c].at[pl.ds(r0, 64)])
            r0 = s * 1000 + 960
            pltpu.sync_copy(agg_sh.at[pl.ds(r0, 40)], vrows.at[pl.ds(0, 40)])
            pltpu.sync_copy(vrows.at[pl.ds(0, 40)],
                            agg_out.at[c].at[pl.ds(r0, 40)])

        @pl.when(s < 8)
        def _():
            for i in range(10):
                r0 = s * 1280 + i * 128
                pltpu.sync_copy(sums_sh.at[pl.ds(r0, 128)], zb1)
                pltpu.sync_copy(zb1, sums_out.at[c].at[pl.ds(r0, 128)])

    return ek(Q, K, V, src3, dst3, eb3)


def _final_body(agg_ref, sums_ref, f_ref, wo_ref, bo_ref, g_ref, b_ref,
                out_ref):
    sums = sums_ref[0] + sums_ref[1]
    inv = 1.0 / jnp.maximum(sums, 1e-8)
    out_nodes = (agg_ref[0] + agg_ref[1]) * inv[:, None]
    output = (
        jnp.dot(out_nodes, wo_ref[...], preferred_element_type=jnp.float32)
        + bo_ref[...]
    )
    res = f_ref[...] + output
    mean = jnp.mean(res, axis=-1, keepdims=True)
    var = jnp.mean((res - mean) ** 2, axis=-1, keepdims=True)
    normed = (res - mean) * lax.rsqrt(var + 1e-5)
    out_ref[...] = normed * g_ref[...] + b_ref[...]


def _final(agg, sums, features, Wo, bo, gamma, beta):
    return pl.pallas_call(
        _final_body,
        out_shape=jax.ShapeDtypeStruct((N, D), jnp.float32),
    )(agg, sums, features, Wo, bo, gamma, beta)


def kernel(features, edge_index, edge_dist, Wq, bq, Wk, bk, Wv, bv,
           pw1, pb1, pw2, pb2, Wo, bo, gamma, beta):
    src = edge_index[0]
    dst = edge_index[1]

    Wcat = jnp.concatenate([Wq, Wk, Wv], axis=1)
    bcat = jnp.concatenate([bq, bk, bv], axis=0)
    Q, K, V, eb2d = _qkv(features, Wcat, bcat, edge_dist, pw1, pb1, pw2, pb2)

    eb = eb2d.reshape(E)

    src3 = src.reshape(JOBS, 1, H)
    dst3 = dst.reshape(JOBS, 1, H)
    eb3 = eb.reshape(JOBS, 1, H)
    agg, sums = _edge_sc(Q, K, V, src3, dst3, eb3)
    sums = sums[:, :N]

    return _final(agg, sums, features, Wo, bo, gamma, beta)


# speedup vs baseline: 1.0108x; 1.0008x over previous
import functools

import jax
import jax.numpy as jnp
import numpy as np
from jax import lax
from jax.experimental import pallas as pl
from jax.experimental.pallas import tpu as pltpu
from jax.experimental.pallas import tpu_sc as plsc

N = 10000
E = 320000
D = 128
NC = 2
NS = 16
NW = NC * NS
H = 64
JOBS = E // H
SLOTS = (JOBS + NW - 1) // NW
NPAD = 10240
INV_SQRT_D = float(1.0 / np.sqrt(D))

_PHI = (1.0 + np.sqrt(5.0)) / 2.0
_HARMONICS = np.array(
    [_PHI / 2, _PHI, _PHI * 1.5, _PHI * _PHI, _PHI * _PHI * 1.5, _PHI ** 3],
    dtype=np.float32,
)


def _qkv_body(f_ref, w_ref, b_ref, dist_ref, pw1_ref, pb1_ref, pw2_ref,
              pb2_ref, q_ref, k_ref, v_ref, eb_ref):
    qkv = (
        jnp.dot(f_ref[...], w_ref[...], preferred_element_type=jnp.float32)
        + b_ref[...]
    )
    q_ref[...] = qkv[:, :D]
    k_ref[...] = qkv[:, D:2 * D]
    v_ref[...] = qkv[:, 2 * D:]
    d = dist_ref[...]
    m = jnp.abs(d - _HARMONICS[0])
    for hk in _HARMONICS[1:]:
        m = jnp.minimum(m, jnp.abs(d - hk))
    w1 = pw1_ref[...]
    b1 = pb1_ref[...]
    w2 = pw2_ref[...]
    acc = jnp.zeros_like(d)
    for j in range(16):
        h = m * w1[0, j] + b1[0, j]
        h = h * jax.nn.sigmoid(h)
        acc = acc + h * w2[0, j]
    pw = jax.nn.sigmoid(acc + pb2_ref[0, 0])
    eb_ref[...] = jnp.exp(pw * 2.0)


def _qkv(features, Wcat, bcat, edge_dist, pw1, pb1, pw2, pb2):
    outs = pl.pallas_call(
        _qkv_body,
        out_shape=[jax.ShapeDtypeStruct((N, D), jnp.float32)] * 3
        + [jax.ShapeDtypeStruct((E // 512, 512), jnp.float32)],
    )(features, Wcat, bcat, edge_dist.reshape(E // 512, 512), pw1,
      pb1.reshape(1, 16), pw2.reshape(1, 16), pb2.reshape(1, 1))
    return outs


def _edge_sc(Q, K, V, src3, dst3, eb3):
    mesh = plsc.VectorSubcoreMesh(core_axis_name="c", subcore_axis_name="s")

    @functools.partial(
        pl.kernel,
        out_type=[
            jax.ShapeDtypeStruct((NC, N, D), jnp.float32),
            jax.ShapeDtypeStruct((NC, NPAD), jnp.float32),
        ],
        mesh=mesh,
        compiler_params=pltpu.CompilerParams(needs_layout_passes=False),
        scratch_types=[
            [pltpu.VMEM((1, H), jnp.int32) for _ in range(4)],
            [pltpu.VMEM((1, H), jnp.int32) for _ in range(4)],
            [pltpu.VMEM((1, H), jnp.float32) for _ in range(4)],
            [pltpu.VMEM((H, D), jnp.float32) for _ in range(2)],
            [pltpu.VMEM((H, D), jnp.float32) for _ in range(2)],
            pltpu.VMEM((H, D), jnp.float32),
            pltpu.VMEM((H * 16,), jnp.float32),
            pltpu.VMEM((H,), jnp.float32),
            pltpu.VMEM((128,), jnp.float32),
            pltpu.VMEM_SHARED((N, D), jnp.float32),
            pltpu.VMEM_SHARED((NPAD,), jnp.float32),
            [pltpu.SemaphoreType.DMA for _ in range(4)],
            [pltpu.SemaphoreType.DMA for _ in range(2)],
            pltpu.SemaphoreType.DMA,
            pltpu.SemaphoreType.DMA,
            pltpu.SemaphoreType.DMA,
        ],
    )
    def ek(q_hbm, k_hbm, v_hbm, src_hbm, dst_hbm, eb_hbm, agg_out, sums_out,
           srch, dsth, ebh, qrows, krows, vrows, attnpart, wbuf, zb1,
           agg_sh, sums_sh, idxsem, qksem, vsem, ssem, wsem):
        c = lax.axis_index("c")
        s = lax.axis_index("s")
        zero16 = jnp.zeros((16,), jnp.float32)
        wid = c * NS + s
        lanes = lax.iota(jnp.int32, 16)

        def zrow(i, carry):
            for j in range(8):
                vrows[i, pl.ds(j * 16, 16)] = zero16
            return carry
        lax.fori_loop(0, H, zrow, 0)
        for i in range(8):
            zb1[pl.ds(i * 16, 16)] = zero16

        @pl.when(s < 10)
        def _():
            for i in range(15):
                pltpu.sync_copy(vrows,
                                agg_sh.at[pl.ds(s * 1000 + i * 64, 64)])
            pltpu.sync_copy(vrows.at[pl.ds(0, 40)],
                            agg_sh.at[pl.ds(s * 1000 + 960, 40)])

        @pl.when(s < 8)
        def _():
            for i in range(10):
                pltpu.sync_copy(zb1, sums_sh.at[pl.ds(s * 1280 + i * 128, 128)])

        plsc.subcore_barrier()

        def jid_of(m):
            return m * NW + wid

        def issue_idx(m, i):
            @pl.when(jid_of(m) < JOBS)
            def _():
                j = jid_of(m)
                pltpu.async_copy(src_hbm.at[j], srch[i], idxsem[i])
                pltpu.async_copy(dst_hbm.at[j], dsth[i], idxsem[i])
                pltpu.async_copy(eb_hbm.at[j], ebh[i], idxsem[i])

        def issue_qk(m, i, p):
            @pl.when(jid_of(m) < JOBS)
            def _():
                j = jid_of(m)
                pltpu.make_async_copy(src_hbm.at[j], srch[i], idxsem[i]).wait()
                pltpu.make_async_copy(dst_hbm.at[j], dsth[i], idxsem[i]).wait()
                pltpu.make_async_copy(eb_hbm.at[j], ebh[i], idxsem[i]).wait()
                pltpu.async_copy(q_hbm.at[dsth[i].at[0]], qrows[p], qksem[p])
                pltpu.async_copy(k_hbm.at[srch[i].at[0]], krows[p], qksem[p])

        def issue_v(m, i):
            @pl.when(jid_of(m) < JOBS)
            def _():
                pltpu.async_copy(v_hbm.at[srch[i].at[0]], vrows, vsem)

        def compute(m, i, p):
            @pl.when(jid_of(m) < JOBS)
            def _():
                pltpu.make_async_copy(
                    q_hbm.at[dsth[i].at[0]], qrows[p], qksem[p]).wait()
                pltpu.make_async_copy(
                    k_hbm.at[srch[i].at[0]], krows[p], qksem[p]).wait()

                qr = qrows[p]
                kr = krows[p]

                @plsc.parallel_loop(0, H, unroll=4)
                def dot_body(e):
                    acc = qr[e, pl.ds(0, 16)] * kr[e, pl.ds(0, 16)]
                    for j in range(1, 8):
                        acc = acc + (qr[e, pl.ds(j * 16, 16)]
                                     * kr[e, pl.ds(j * 16, 16)])
                    attnpart[pl.ds(e * 16, 16)] = acc

                @plsc.parallel_loop(0, H // 16, unroll=2)
                def red_body(g):
                    gbase = g * 256
                    parts = [plsc.load_gather(attnpart, [gbase + lanes * 16 + j])
                             for j in range(16)]
                    while len(parts) > 1:
                        parts = [a + b for a, b in zip(parts[::2], parts[1::2])]
                    attn16 = parts[0] * INV_SQRT_D
                    w16 = jnp.exp(attn16) * ebh[i][0, pl.ds(g * 16, 16)]
                    wbuf[pl.ds(g * 16, 16)] = w16

                pltpu.make_async_copy(
                    v_hbm.at[srch[i].at[0]], vrows, vsem).wait()

                @plsc.parallel_loop(0, H, unroll=4)
                def scale_body(e):
                    wsp = plsc.load_gather(
                        wbuf, [jnp.zeros((16,), jnp.int32) + e])
                    for j in range(8):
                        vrows[e, pl.ds(j * 16, 16)] = (
                            vrows[e, pl.ds(j * 16, 16)] * wsp)

                cs = pltpu.async_copy(
                    vrows, agg_sh.at[dsth[i].at[0]], ssem, add=True)
                cw = pltpu.async_copy(
                    wbuf, sums_sh.at[dsth[i].at[0]], wsem, add=True)
                cs.wait()
                cw.wait()

        for i in range(4):
            issue_idx(i, i)
        issue_qk(0, 0, 0)
        issue_v(0, 0)

        def outer(k2, carry):
            kb = k2 * 4
            for u in range(4):
                m = kb + u
                issue_qk(m + 1, (u + 1) % 4, (u + 1) % 2)
                compute(m, u, u % 2)
                issue_v(m + 1, (u + 1) % 4)
                issue_idx(m + 4, u)
            return carry
        lax.fori_loop(0, (SLOTS + 3) // 4, outer, 0)

        plsc.subcore_barrier()

        @pl.when(s < 10)
        def _():
            for i in range(15):
                r0 = s * 1000 + i * 64
                pltpu.sync_copy(agg_sh.at[pl.ds(r0, 64)], vrows)
                pltpu.sync_copy(vrows, agg_out.at[c].at[pl.ds(r0, 64)])
            r0 = s * 1000 + 960
            pltpu.sync_copy(agg_sh.at[pl.ds(r0, 40)], vrows.at[pl.ds(0, 40)])
            pltpu.sync_copy(vrows.at[pl.ds(0, 40)],
                            agg_out.at[c].at[pl.ds(r0, 40)])

        @pl.when(s < 8)
        def _():
            for i in range(10):
                r0 = s * 1280 + i * 128
                pltpu.sync_copy(sums_sh.at[pl.ds(r0, 128)], zb1)
                pltpu.sync_copy(zb1, sums_out.at[c].at[pl.ds(r0, 128)])

    return ek(Q, K, V, src3, dst3, eb3)


def _final_body(agg_ref, sums_ref, f_ref, wo_ref, bo_ref, g_ref, b_ref,
                out_ref):
    sums = sums_ref[0] + sums_ref[1]
    inv = 1.0 / jnp.maximum(sums, 1e-8)
    out_nodes = (agg_ref[0] + agg_ref[1]) * inv[:, None]
    output = (
        jnp.dot(out_nodes, wo_ref[...], preferred_element_type=jnp.float32)
        + bo_ref[...]
    )
    res = f_ref[...] + output
    mean = jnp.mean(res, axis=-1, keepdims=True)
    var = jnp.mean((res - mean) ** 2, axis=-1, keepdims=True)
    normed = (res - mean) * lax.rsqrt(var + 1e-5)
    out_ref[...] = normed * g_ref[...] + b_ref[...]


def _final(agg, sums, features, Wo, bo, gamma, beta):
    return pl.pallas_call(
        _final_body,
        out_shape=jax.ShapeDtypeStruct((N, D), jnp.float32),
    )(agg, sums, features, Wo, bo, gamma, beta)


def kernel(features, edge_index, edge_dist, Wq, bq, Wk, bk, Wv, bv,
           pw1, pb1, pw2, pb2, Wo, bo, gamma, beta):
    src = edge_index[0]
    dst = edge_index[1]

    Wcat = jnp.concatenate([Wq, Wk, Wv], axis=1)
    bcat = jnp.concatenate([bq, bk, bv], axis=0)
    Q, K, V, eb2d = _qkv(features, Wcat, bcat, edge_dist, pw1, pb1, pw2, pb2)

    eb = eb2d.reshape(E)

    src3 = src.reshape(JOBS, 1, H)
    dst3 = dst.reshape(JOBS, 1, H)
    eb3 = eb.reshape(JOBS, 1, H)
    agg, sums = _edge_sc(Q, K, V, src3, dst3, eb3)
    sums = sums[:, :N]

    return _final(agg, sums, features, Wo, bo, gamma, beta)
